# baseline (speedup 1.0000x reference)
import functools

import jax
import jax.numpy as jnp
from jax import lax
from jax.experimental import pallas as pl
from jax.experimental.pallas import tpu as pltpu
from jax.experimental.pallas import tpu_sc as plsc

_NUM_WORKERS = 32
_CHUNK = 160
_SLOTS = 4


def _sc_gather(idx_flat, weight):
    n = idx_flat.shape[0]
    d = weight.shape[1]
    per_w = n // _NUM_WORKERS
    n_chunks = per_w // _CHUNK
    n_groups = n_chunks // _SLOTS
    assert n_chunks % _SLOTS == 0 and n_groups >= 3

    mesh = plsc.VectorSubcoreMesh(core_axis_name="c", subcore_axis_name="s")

    @functools.partial(
        pl.kernel,
        out_type=jax.ShapeDtypeStruct((n, d), jnp.float32),
        mesh=mesh,
        scratch_types=[
            pltpu.VMEM_SHARED(weight.shape, jnp.float32),
            pltpu.VMEM((per_w,), jnp.int32),
            pltpu.VMEM((_SLOTS, _CHUNK, d), jnp.float32),
            [pltpu.SemaphoreType.DMA] * _SLOTS,
            [pltpu.SemaphoreType.DMA] * _SLOTS,
        ],
    )
    def k(idx_hbm, table_hbm, out_hbm, table_sh, idx_v, rows_v, gsems, osems):
        sid = lax.axis_index("s")
        wid = sid * 2 + lax.axis_index("c")
        base = pl.multiple_of(wid * per_w, 8)

        @pl.when(sid == 0)
        def _():
            pltpu.sync_copy(table_hbm, table_sh)

        pltpu.sync_copy(idx_hbm.at[pl.ds(base, per_w)], idx_v)
        plsc.subcore_barrier()

        def gstart(g, b):
            off = pl.multiple_of(g * _CHUNK, 8)
            pltpu.async_copy(
                table_sh.at[idx_v.at[pl.ds(off, _CHUNK)]], rows_v.at[b], gsems[b]
            )

        def gwait(b):
            pltpu.make_async_copy(
                table_sh.at[idx_v.at[pl.ds(0, _CHUNK)]], rows_v.at[b], gsems[b]
            ).wait()

        def ostart(g, b):
            off = pl.multiple_of(base + g * _CHUNK, 8)
            pltpu.async_copy(
                rows_v.at[b], out_hbm.at[pl.ds(off, _CHUNK)], osems[b]
            )

        def owait(b):
            pltpu.make_async_copy(
                rows_v.at[b], out_hbm.at[pl.ds(base, _CHUNK)], osems[b]
            ).wait()

        gstart(0, 0)
        for b in range(_SLOTS):
            gwait(b)
            if b < _SLOTS - 1:
                gstart(b + 1, b + 1)
            else:
                owait(0)
                gstart(_SLOTS, 0)
            ostart(b, b)

        def body(i, carry):
            t = i * _SLOTS
            for b in range(_SLOTS):
                g = t + b
                bn = (b + 1) % _SLOTS
                gwait(b)
                owait(bn)
                gstart(g + 1, bn)
                ostart(g, b)
            return carry

        lax.fori_loop(1, n_groups - 1, body, 0)

        t = n_chunks - _SLOTS
        for b in range(_SLOTS):
            g = t + b
            gwait(b)
            if b < _SLOTS - 1:
                owait(b + 1)
                gstart(g + 1, b + 1)
            ostart(g, b)
        for b in range(_SLOTS):
            owait(b)

    return k(idx_flat, weight)


def kernel(label_ids, weight):
    b, l = label_ids.shape
    d = weight.shape[1]
    idx_flat = label_ids.reshape(-1).astype(jnp.int32)
    out = _sc_gather(idx_flat, weight)
    return out.reshape(b, l, d)


# speedup vs baseline: 1.0241x; 1.0241x over previous
import functools

import jax
import jax.numpy as jnp
from jax import lax
from jax.experimental import pallas as pl
from jax.experimental.pallas import tpu as pltpu
from jax.experimental.pallas import tpu_sc as plsc

_NUM_WORKERS = 32
_CHUNK = 320
_SLOTS = 2


def _sc_gather(idx_flat, weight):
    n = idx_flat.shape[0]
    d = weight.shape[1]
    per_w = n // _NUM_WORKERS
    n_chunks = per_w // _CHUNK
    n_groups = n_chunks // _SLOTS
    assert n_chunks % _SLOTS == 0 and n_groups >= 3

    mesh = plsc.VectorSubcoreMesh(core_axis_name="c", subcore_axis_name="s")

    @functools.partial(
        pl.kernel,
        out_type=jax.ShapeDtypeStruct((n, d), jnp.float32),
        mesh=mesh,
        scratch_types=[
            pltpu.VMEM_SHARED(weight.shape, jnp.float32),
            pltpu.VMEM((per_w,), jnp.int32),
            pltpu.VMEM((_SLOTS, _CHUNK, d), jnp.float32),
            [pltpu.SemaphoreType.DMA] * _SLOTS,
            [pltpu.SemaphoreType.DMA] * _SLOTS,
        ],
    )
    def k(idx_hbm, table_hbm, out_hbm, table_sh, idx_v, rows_v, gsems, osems):
        sid = lax.axis_index("s")
        wid = sid * 2 + lax.axis_index("c")
        base = pl.multiple_of(wid * per_w, 8)

        @pl.when(sid == 0)
        def _():
            pltpu.sync_copy(table_hbm, table_sh)

        pltpu.sync_copy(idx_hbm.at[pl.ds(base, per_w)], idx_v)
        plsc.subcore_barrier()

        def gstart(g, b):
            off = pl.multiple_of(g * _CHUNK, 8)
            pltpu.async_copy(
                table_sh.at[idx_v.at[pl.ds(off, _CHUNK)]], rows_v.at[b], gsems[b]
            )

        def gwait(b):
            pltpu.make_async_copy(
                table_sh.at[idx_v.at[pl.ds(0, _CHUNK)]], rows_v.at[b], gsems[b]
            ).wait()

        def ostart(g, b):
            off = pl.multiple_of(base + g * _CHUNK, 8)
            pltpu.async_copy(
                rows_v.at[b], out_hbm.at[pl.ds(off, _CHUNK)], osems[b]
            )

        def owait(b):
            pltpu.make_async_copy(
                rows_v.at[b], out_hbm.at[pl.ds(base, _CHUNK)], osems[b]
            ).wait()

        gstart(0, 0)
        for b in range(_SLOTS):
            gwait(b)
            if b < _SLOTS - 1:
                gstart(b + 1, b + 1)
            else:
                owait(0)
                gstart(_SLOTS, 0)
            ostart(b, b)

        def body(i, carry):
            t = i * _SLOTS
            for b in range(_SLOTS):
                g = t + b
                bn = (b + 1) % _SLOTS
                gwait(b)
                owait(bn)
                gstart(g + 1, bn)
                ostart(g, b)
            return carry

        lax.fori_loop(1, n_groups - 1, body, 0)

        t = n_chunks - _SLOTS
        for b in range(_SLOTS):
            g = t + b
            gwait(b)
            if b < _SLOTS - 1:
                owait(b + 1)
                gstart(g + 1, b + 1)
            ostart(g, b)
        for b in range(_SLOTS):
            owait(b)

    return k(idx_flat, weight)


def kernel(label_ids, weight):
    b, l = label_ids.shape
    d = weight.shape[1]
    idx_flat = label_ids.reshape(-1).astype(jnp.int32)
    out = _sc_gather(idx_flat, weight)
    return out.reshape(b, l, d)


# speedup vs baseline: 1.0269x; 1.0027x over previous
import functools

import jax
import jax.numpy as jnp
from jax import lax
from jax.experimental import pallas as pl
from jax.experimental.pallas import tpu as pltpu
from jax.experimental.pallas import tpu_sc as plsc

_NUM_WORKERS = 32
_CHUNK = 400


def _sc_gather(idx_flat, weight):
    n = idx_flat.shape[0]
    d = weight.shape[1]
    per_w = n // _NUM_WORKERS
    nc = per_w // _CHUNK
    assert nc % 2 == 0 and nc >= 6

    mesh = plsc.VectorSubcoreMesh(core_axis_name="c", subcore_axis_name="s")

    @functools.partial(
        pl.kernel,
        out_type=jax.ShapeDtypeStruct((n, d), jnp.float32),
        mesh=mesh,
        scratch_types=[
            pltpu.VMEM_SHARED(weight.shape, jnp.float32),
            pltpu.VMEM((_CHUNK,), jnp.int32),
            pltpu.VMEM((_CHUNK,), jnp.int32),
            pltpu.VMEM((2, _CHUNK, d), jnp.float32),
            [pltpu.SemaphoreType.DMA] * 2,
            [pltpu.SemaphoreType.DMA] * 2,
            [pltpu.SemaphoreType.DMA] * 2,
        ],
    )
    def k(idx_hbm, table_hbm, out_hbm, table_sh, idx_v0, idx_v1, rows_v,
          isems, gsems, osems):
        sid = lax.axis_index("s")
        wid = sid * 2 + lax.axis_index("c")
        base = pl.multiple_of(wid * per_w, 8)

        @pl.when(sid == 0)
        def _():
            pltpu.sync_copy(table_hbm, table_sh)

        plsc.subcore_barrier()

        idx_bufs = (idx_v0, idx_v1)

        def ixstart(g, b):
            off = pl.multiple_of(base + g * _CHUNK, 8)
            pltpu.async_copy(idx_hbm.at[pl.ds(off, _CHUNK)], idx_bufs[b],
                             isems[b])

        def ixwait(b):
            pltpu.make_async_copy(
                idx_hbm.at[pl.ds(base, _CHUNK)], idx_bufs[b], isems[b]
            ).wait()

        def gstart(b):
            pltpu.async_copy(
                table_sh.at[idx_bufs[b]], rows_v.at[b], gsems[b]
            )

        def gwait(b):
            pltpu.make_async_copy(
                table_sh.at[idx_bufs[b]], rows_v.at[b], gsems[b]
            ).wait()

        def ostart(g, b):
            off = pl.multiple_of(base + g * _CHUNK, 8)
            pltpu.async_copy(
                rows_v.at[b], out_hbm.at[pl.ds(off, _CHUNK)], osems[b]
            )

        def owait(b):
            pltpu.make_async_copy(
                rows_v.at[b], out_hbm.at[pl.ds(base, _CHUNK)], osems[b]
            ).wait()


        ixstart(0, 0)
        ixwait(0)
        gstart(0)
        ixstart(1, 1)

        gwait(0)
        ostart(0, 0)
        ixstart(2, 0)
        ixwait(1)
        gstart(1)

        def body(i, carry):
            for b in (1, 0):
                g = 2 * i - 1 + (1 - b)
                bn = 1 - b
                gwait(b)
                ostart(g, b)
                ixstart(g + 2, b)
                ixwait(bn)
                owait(bn)
                gstart(bn)
            return carry

        lax.fori_loop(1, (nc - 2) // 2, body, 0)

        gwait(1)
        ostart(nc - 3, 1)
        ixstart(nc - 1, 1)
        ixwait(0)
        owait(0)
        gstart(0)

        gwait(0)
        ostart(nc - 2, 0)
        ixwait(1)
        owait(1)
        gstart(1)

        gwait(1)
        ostart(nc - 1, 1)

        owait(0)
        owait(1)

    return k(idx_flat, weight)


def kernel(label_ids, weight):
    b, l = label_ids.shape
    d = weight.shape[1]
    idx_flat = label_ids.reshape(-1).astype(jnp.int32)
    out = _sc_gather(idx_flat, weight)
    return out.reshape(b, l, d)


# speedup vs baseline: 1.0280x; 1.0010x over previous
import functools

import jax
import jax.numpy as jnp
from jax import lax
from jax.experimental import pallas as pl
from jax.experimental.pallas import tpu as pltpu
from jax.experimental.pallas import tpu_sc as plsc

_NUM_WORKERS = 32
_CHUNK = 400


def _sc_gather(idx_flat, weight):
    n = idx_flat.shape[0]
    d = weight.shape[1]
    per_w = n // _NUM_WORKERS
    nc = per_w // _CHUNK
    assert nc % 2 == 0 and nc >= 6

    mesh = plsc.VectorSubcoreMesh(core_axis_name="c", subcore_axis_name="s")

    @functools.partial(
        pl.kernel,
        out_type=jax.ShapeDtypeStruct((n, d), jnp.float32),
        mesh=mesh,
        scratch_types=[
            pltpu.VMEM_SHARED(weight.shape, jnp.float32),
            pltpu.VMEM((_CHUNK,), jnp.int32),
            pltpu.VMEM((_CHUNK,), jnp.int32),
            pltpu.VMEM((2, _CHUNK, d), jnp.float32),
            [pltpu.SemaphoreType.DMA] * 2,
            [pltpu.SemaphoreType.DMA] * 2,
            [pltpu.SemaphoreType.DMA] * 2,
        ],
    )
    def k(idx_hbm, table_hbm, out_hbm, table_sh, idx_v0, idx_v1, rows_v,
          isems, gsems, osems):
        sid = lax.axis_index("s")
        wid = sid * 2 + lax.axis_index("c")
        base = pl.multiple_of(wid * per_w, 8)

        idx_bufs = (idx_v0, idx_v1)

        def ixstart(g, b):
            off = pl.multiple_of(base + g * _CHUNK, 8)
            pltpu.async_copy(idx_hbm.at[pl.ds(off, _CHUNK)], idx_bufs[b],
                             isems[b])

        def ixwait(b):
            pltpu.make_async_copy(
                idx_hbm.at[pl.ds(base, _CHUNK)], idx_bufs[b], isems[b]
            ).wait()

        def gstart(b):
            pltpu.async_copy(
                table_sh.at[idx_bufs[b]], rows_v.at[b], gsems[b]
            )

        def gwait(b):
            pltpu.make_async_copy(
                table_sh.at[idx_bufs[b]], rows_v.at[b], gsems[b]
            ).wait()

        def ostart(g, b):
            off = pl.multiple_of(base + g * _CHUNK, 8)
            pltpu.async_copy(
                rows_v.at[b], out_hbm.at[pl.ds(off, _CHUNK)], osems[b]
            )

        def owait(b):
            pltpu.make_async_copy(
                rows_v.at[b], out_hbm.at[pl.ds(base, _CHUNK)], osems[b]
            ).wait()


        ixstart(0, 0)
        ixstart(1, 1)

        @pl.when(sid == 0)
        def _():
            pltpu.sync_copy(table_hbm, table_sh)

        plsc.subcore_barrier()

        ixwait(0)
        gstart(0)

        gwait(0)
        ostart(0, 0)
        ixstart(2, 0)
        ixwait(1)
        gstart(1)

        def body(i, carry):
            for b in (1, 0):
                g = 2 * i - 1 + (1 - b)
                bn = 1 - b
                gwait(b)
                ostart(g, b)
                ixstart(g + 2, b)
                ixwait(bn)
                owait(bn)
                gstart(bn)
            return carry

        lax.fori_loop(1, (nc - 2) // 2, body, 0)

        gwait(1)
        ostart(nc - 3, 1)
        ixstart(nc - 1, 1)
        ixwait(0)
        owait(0)
        gstart(0)

        gwait(0)
        ostart(nc - 2, 0)
        ixwait(1)
        owait(1)
        gstart(1)

        gwait(1)
        ostart(nc - 1, 1)

        owait(0)
        owait(1)

    return k(idx_flat, weight)


def kernel(label_ids, weight):
    b, l = label_ids.shape
    d = weight.shape[1]
    idx_flat = label_ids.reshape(-1).astype(jnp.int32)
    out = _sc_gather(idx_flat, weight)
    return out.reshape(b, l, d)
